# baseline (speedup 1.0000x reference)
import functools
import jax
import jax.numpy as jnp
from jax import lax
from jax.experimental import pallas as pl
from jax.experimental.pallas import tpu as pltpu
from jax.experimental.pallas import tpu_sc as plsc

_info = plsc.get_sparse_core_info()
_NC, _NS = _info.num_cores, _info.num_subcores
_NW = _NC * _NS
_NBUF = 8


@functools.partial(jax.jit, static_argnames=("n_batch",))
def _sc_gather(idx3d, table, n_batch):
    S = idx3d.shape[2]
    D = table.shape[1]
    slabs_per_w = n_batch // _NW
    n_groups = slabs_per_w // _NBUF
    mesh = plsc.VectorSubcoreMesh(core_axis_name="c", subcore_axis_name="s")

    @functools.partial(
        pl.kernel,
        mesh=mesh,
        out_type=jax.ShapeDtypeStruct((n_batch, S, D), jnp.float32),
        scratch_types=[
            pltpu.VMEM((slabs_per_w, S), jnp.int32),
            pltpu.VMEM((_NBUF, S, D), jnp.float32),
        ]
        + [pltpu.SemaphoreType.DMA] * (2 * _NBUF),
    )
    def k(idx_hbm, table_hbm, out_hbm, idx_v, rows_v, *sems):
        gsems, ssems = sems[:_NBUF], sems[_NBUF:]
        wid = lax.axis_index("s") * _NC + lax.axis_index("c")
        base_b = wid * slabs_per_w
        pltpu.sync_copy(idx_hbm.at[wid], idx_v)

        def gather(j, b):
            pltpu.async_copy(table_hbm.at[idx_v.at[j]], rows_v.at[b], gsems[b])

        for b in range(_NBUF):
            gather(b, b)

        def group(p, carry):
            j0 = p * _NBUF
            for b in range(_NBUF):
                pltpu.make_async_copy(
                    table_hbm.at[idx_v.at[b]], rows_v.at[b], gsems[b]
                ).wait()
                pltpu.async_copy(rows_v.at[b], out_hbm.at[base_b + j0 + b], ssems[b])
            for b in range(_NBUF):
                pltpu.make_async_copy(
                    rows_v.at[b], out_hbm.at[base_b + j0 + b], ssems[b]
                ).wait()

                @pl.when(p + 1 < n_groups)
                def _():
                    gather(j0 + b + _NBUF, b)

            return carry

        lax.fori_loop(0, n_groups, group, 0)

    return k(idx3d, table)


def kernel(token_ids, weight):
    n_batch, S = token_ids.shape
    ids = token_ids.astype(jnp.int32)
    granule = _NW * _NBUF
    pad = (-n_batch) % granule
    if pad:
        ids = jnp.concatenate([ids, jnp.zeros((pad, S), jnp.int32)])
    idx3d = ids.reshape(_NW, (n_batch + pad) // _NW, S)
    out = _sc_gather(idx3d, weight, n_batch + pad)
    if pad:
        out = out[:n_batch]
    return out
